# Initial kernel scaffold; baseline (speedup 1.0000x reference)
import jax
import jax.numpy as jnp
from jax.experimental import pallas as pl


def kernel(output, target):
    raise NotImplementedError("write your pallas kernel here")


# speedup vs baseline: 115.5949x; 115.5949x over previous
import functools

import jax
import jax.numpy as jnp
from jax.experimental import pallas as pl
from jax.experimental.pallas import tpu as pltpu

N_MASK = 1024


def _bce(x, t):
    return jnp.maximum(x, 0.0) - x * t + jnp.log1p(jnp.exp(-jnp.abs(x)))


def _topk_sum_kernel(out_ref, tgt_ref, res_ref):
    x = out_ref[...]
    t = tgt_ref[...]
    loss = _bce(x, t)
    bits = loss.view(jnp.int32)

    def body(i, cur):
        b = 30 - i
        trial = cur | (jnp.int32(1) << b)
        cnt = jnp.sum((bits >= trial).astype(jnp.int32), axis=1, keepdims=True)
        return jnp.where(cnt >= N_MASK, trial, cur)

    kth = jax.lax.fori_loop(0, 31, body, jnp.zeros((loss.shape[0], 1), jnp.int32))
    thr = kth.view(jnp.float32)
    gt = loss > thr
    sum_gt = jnp.sum(jnp.where(gt, loss, 0.0), axis=1, keepdims=True)
    cnt_gt = jnp.sum(gt.astype(jnp.float32), axis=1, keepdims=True)
    row_sum = sum_gt + (N_MASK - cnt_gt) * thr
    res_ref[0, 0] = jnp.sum(row_sum)


@jax.jit
def kernel(output, target):
    bs = output.shape[0]
    res = pl.pallas_call(
        _topk_sum_kernel,
        out_shape=jax.ShapeDtypeStruct((1, 1), jnp.float32),
        out_specs=pl.BlockSpec(memory_space=pltpu.SMEM),
    )(output, target)
    return (res[0, 0] / (bs * N_MASK)).astype(jnp.float32)
